# Initial kernel scaffold; baseline (speedup 1.0000x reference)
import jax
import jax.numpy as jnp
from jax.experimental import pallas as pl


def kernel(x, table, W, b):
    raise NotImplementedError("write your pallas kernel here")


# speedup vs baseline: 1.5291x; 1.5291x over previous
import functools

import jax
import jax.numpy as jnp
from jax import lax
from jax.experimental import pallas as pl
from jax.experimental.pallas import tpu as pltpu
from jax.experimental.pallas import tpu_sc as plsc

BATCH = 16384
SEQ = 20
EMBED_DIM = 32
VOCAB_P1 = 1000001

NUM_WORKERS = 32
SAMPLES_PER_WORKER = BATCH // NUM_WORKERS
CHUNK = 64
NCHUNK = SAMPLES_PER_WORKER // CHUNK
ROWS_PER_CHUNK = CHUNK * SEQ
GATHER_BLK = 128
NGATHER = ROWS_PER_CHUNK // GATHER_BLK


def _sc_body(xf_hbm, table_hbm, wb_hbm, out_hbm,
             idx_v, rows_v, wb_v, row0_v, out_v, sem):
    nc = 2
    wid = lax.axis_index("s") * nc + lax.axis_index("c")
    base_s = wid * SAMPLES_PER_WORKER

    pltpu.sync_copy(wb_hbm, wb_v)
    bv = wb_v[pl.ds(32, 16)]

    pltpu.sync_copy(table_hbm.at[pl.ds(0, 1)],
                    row0_v.at[pl.ds(0, 1), pl.ds(0, EMBED_DIM)])

    lane = lax.iota(jnp.int32, 16)

    def chunk_body(c, carry):
        flat_off = (base_s + c * CHUNK) * SEQ
        pltpu.sync_copy(xf_hbm.at[pl.ds(flat_off, ROWS_PER_CHUNK)], idx_v)

        def fire(j, _):
            sl = pl.ds(j * GATHER_BLK, GATHER_BLK)
            pltpu.async_copy(table_hbm.at[idx_v.at[sl]], rows_v.at[sl], sem)
            return 0
        lax.fori_loop(0, NGATHER, fire, 0)
        for _ in range(NGATHER):
            pltpu.make_async_copy(
                table_hbm.at[idx_v.at[pl.ds(0, GATHER_BLK)]],
                rows_v.at[pl.ds(0, GATHER_BLK)], sem).wait()

        def group_body(g, _):
            row_base = g * (16 * SEQ) + lane * SEQ
            row_vecs = [row_base + p for p in range(SEQ)]

            cnt = jnp.zeros((16,), jnp.float32)
            for p in range(SEQ):
                v = plsc.load_gather(idx_v, [row_vecs[p]])
                cnt = cnt + jnp.where(v == 0, 1.0, 0.0).astype(jnp.float32)

            def dim_body(d, acc):
                col = jnp.full((16,), d, jnp.int32)
                tmp = jnp.zeros((16,), jnp.float32)
                for p in range(SEQ):
                    tmp = tmp + plsc.load_gather(rows_v, [row_vecs[p], col])
                w_d = wb_v[pl.ds(d, 16)][0]
                t0_d = row0_v[0, pl.ds(d, 16)][0]
                return acc + (tmp - cnt * t0_d) * w_d
            sums = lax.fori_loop(0, EMBED_DIM, dim_body,
                                 jnp.zeros((16,), jnp.float32))

            logits = sums * (1.0 / SEQ) + bv
            probs = 1.0 / (1.0 + jnp.exp(-logits))
            out_v[pl.ds(c * CHUNK + g * 16, 16)] = probs
            return 0
        lax.fori_loop(0, CHUNK // 16, group_body, 0)
        return carry

    lax.fori_loop(0, NCHUNK, chunk_body, 0)
    pltpu.sync_copy(out_v, out_hbm.at[pl.ds(base_s, SAMPLES_PER_WORKER)])


@jax.jit
def _run(xf, table, wb):
    mesh = plsc.VectorSubcoreMesh(core_axis_name="c", subcore_axis_name="s")
    f = functools.partial(
        pl.kernel, mesh=mesh,
        out_type=jax.ShapeDtypeStruct((BATCH,), jnp.float32),
        scratch_types=[
            pltpu.VMEM((ROWS_PER_CHUNK,), jnp.int32),
            pltpu.VMEM((ROWS_PER_CHUNK, EMBED_DIM), jnp.float32),
            pltpu.VMEM((48,), jnp.float32),
            pltpu.VMEM((1, 48), jnp.float32),
            pltpu.VMEM((SAMPLES_PER_WORKER,), jnp.float32),
            pltpu.SemaphoreType.DMA,
        ],
        compiler_params=pltpu.CompilerParams(
            needs_layout_passes=False, use_tc_tiling_on_sc=False),
    )(_sc_body)
    return f(xf, table, wb)


def kernel(x, table, W, b):
    xf = x.reshape(-1).astype(jnp.int32)
    wb = jnp.concatenate(
        [W.reshape(-1).astype(jnp.float32),
         jnp.full((16,), b[0], jnp.float32)])
    out = _run(xf, table, wb)
    return out.reshape(BATCH, 1)
